# baseline (speedup 1.0000x reference)
import functools

import jax
import jax.numpy as jnp
from jax import lax
from jax.experimental import pallas as pl
from jax.experimental.pallas import tpu as pltpu
from jax.experimental.pallas import tpu_sc as plsc

B = 4
S = 2048
D = 2048
N_ROUTES = 16
NVT = 100

S_BLK = 256
NS_BLK = S // S_BLK

NC = 2
NSUB = 16
NW = NC * NSUB
ROWS_PER_W = 13
ROWS_PAD = NW * ROWS_PER_W
SLOTS = 2 * ROWS_PER_W
SLOT_PAD = 32
LANES = 16


def _route_body(x_ref, mb_ref, mfull_ref, wr_ref, g_ref, w_ref, acc_ref):
    s = pl.program_id(0)
    xb = x_ref[...]
    mb = mb_ref[...].astype(jnp.float32)
    part = jnp.sum(xb * mb[:, :, None], axis=1)

    @pl.when(s == 0)
    def _init():
        acc_ref[...] = part

    @pl.when(s > 0)
    def _accum():
        acc_ref[...] = acc_ref[...] + part

    @pl.when(s == NS_BLK - 1)
    def _finalize():
        mfull = mfull_ref[...].astype(jnp.float32)
        cnt = jnp.clip(jnp.sum(mfull, axis=1, keepdims=True), 1.0, None)
        sent = acc_ref[...] / cnt
        logits = lax.dot_general(
            sent, wr_ref[...], (((1,), (1,)), ((), ())),
            precision=lax.Precision.HIGHEST,
            preferred_element_type=jnp.float32)
        z = logits - jnp.max(logits, axis=1, keepdims=True)
        ez = jnp.exp(z)
        p = ez / jnp.sum(ez, axis=1, keepdims=True)

        iota = lax.broadcasted_iota(jnp.int32, (B, N_ROUTES), 1)
        m1 = jnp.max(p, axis=1, keepdims=True)
        i1 = jnp.min(jnp.where(p == m1, iota, N_ROUTES), axis=1, keepdims=True)
        p2 = jnp.where(iota == i1, -1.0, p)
        m2 = jnp.max(p2, axis=1, keepdims=True)
        i2 = jnp.min(jnp.where(p2 == m2, iota, N_ROUTES), axis=1, keepdims=True)

        wq = lax.broadcasted_iota(jnp.int32, (NW, SLOT_PAD), 0)
        sq = lax.broadcasted_iota(jnp.int32, (NW, SLOT_PAD), 1)
        lq = sq % LANES
        route0 = sq < LANES
        r = wq * ROWS_PER_W + lq
        bq = jnp.minimum(r // NVT, B - 1)
        jq = r % NVT
        valid = (lq < ROWS_PER_W) & (r < B * NVT)
        gsel = jnp.zeros((NW, SLOT_PAD), jnp.int32)
        wsel = jnp.zeros((NW, SLOT_PAD), jnp.float32)
        for bb in range(B):
            onb = bq == bb
            t1 = lax.slice(i1, (bb, 0), (bb + 1, 1))
            t2 = lax.slice(i2, (bb, 0), (bb + 1, 1))
            v1 = lax.slice(m1, (bb, 0), (bb + 1, 1))
            v2 = lax.slice(m2, (bb, 0), (bb + 1, 1))
            gsel = gsel + jnp.where(onb, jnp.where(route0, t1, t2), 0)
            wsel = wsel + jnp.where(onb, jnp.where(route0, v1, v2), 0.0)
        g_ref[...] = gsel * NVT + jq
        w_ref[...] = jnp.where(valid, wsel, 0.0)


_route = pl.pallas_call(
    _route_body,
    grid=(NS_BLK,),
    in_specs=[
        pl.BlockSpec((B, S_BLK, D), lambda s: (0, s, 0)),
        pl.BlockSpec((B, S_BLK), lambda s: (0, s)),
        pl.BlockSpec((B, S), lambda s: (0, 0)),
        pl.BlockSpec((N_ROUTES, D), lambda s: (0, 0)),
    ],
    out_specs=[
        pl.BlockSpec((NW, SLOT_PAD), lambda s: (0, 0)),
        pl.BlockSpec((NW, SLOT_PAD), lambda s: (0, 0)),
    ],
    out_shape=[
        jax.ShapeDtypeStruct((NW, SLOT_PAD), jnp.int32),
        jax.ShapeDtypeStruct((NW, SLOT_PAD), jnp.float32),
    ],
    scratch_shapes=[pltpu.VMEM((B, D), jnp.float32)],
)


def _combine_body(emb_ref, g_ref, w_ref, out_ref, g_v, w_v, rows_v, out_v, sem):
    wid = lax.axis_index("s") * NC + lax.axis_index("c")
    pltpu.sync_copy(g_ref, g_v)
    pltpu.sync_copy(w_ref, w_v)
    pltpu.async_copy(emb_ref.at[g_v.at[wid]], rows_v, sem).wait()
    wa = w_v[wid, pl.ds(0, LANES)]
    wb = w_v[wid, pl.ds(LANES, LANES)]

    dnums = lax.GatherDimensionNumbers(
        offset_dims=(), collapsed_slice_dims=(0,), start_index_map=(0,))

    def row_body(i, carry):
        iv = jnp.full((LANES, 1), i, jnp.int32)
        w0 = lax.gather(wa, iv, dnums, (1,),
                        mode=lax.GatherScatterMode.PROMISE_IN_BOUNDS)
        w1 = lax.gather(wb, iv, dnums, (1,),
                        mode=lax.GatherScatterMode.PROMISE_IN_BOUNDS)

        def col_body(c, carry2):
            a = rows_v[i, pl.ds(c * LANES, LANES)]
            b2 = rows_v[i + LANES, pl.ds(c * LANES, LANES)]
            out_v[i, pl.ds(c * LANES, LANES)] = a * w0 + b2 * w1
            return carry2

        return lax.fori_loop(0, D // LANES, col_body, carry)

    lax.fori_loop(0, ROWS_PER_W, row_body, 0)
    pltpu.sync_copy(out_v, out_ref.at[wid])


@functools.cache
def _get_combine():
    return pl.kernel(
        _combine_body,
        out_type=jax.ShapeDtypeStruct((NW, ROWS_PER_W, D), jnp.float32),
        mesh=plsc.VectorSubcoreMesh(core_axis_name="c", subcore_axis_name="s",
                                    num_cores=NC, num_subcores=NSUB),
        scratch_types=[
            pltpu.VMEM((NW, SLOT_PAD), jnp.int32),
            pltpu.VMEM((NW, SLOT_PAD), jnp.float32),
            pltpu.VMEM((SLOT_PAD, D), jnp.float32),
            pltpu.VMEM((ROWS_PER_W, D), jnp.float32),
            pltpu.SemaphoreType.DMA,
        ],
    )


def kernel(indices, input_ids, inputs_embeds, attention_mask, embedding, W_router):
    g_tab, w_tab = _route(inputs_embeds, attention_mask, attention_mask, W_router)
    out = _get_combine()(embedding, g_tab, w_tab)
    return out.reshape(ROWS_PAD, D)[: B * NVT].reshape(B, NVT, D)


# speedup vs baseline: 1.1228x; 1.1228x over previous
import functools

import jax
import jax.numpy as jnp
from jax import lax
from jax.experimental import pallas as pl
from jax.experimental.pallas import tpu as pltpu
from jax.experimental.pallas import tpu_sc as plsc

B = 4
S = 2048
D = 2048
N_ROUTES = 16
NVT = 100

S_BLK = 256
NS_BLK = S // S_BLK

NC = 2
NSUB = 16
NW = NC * NSUB
ROWS_PER_W = 13
ROWS_PAD = NW * ROWS_PER_W
SLOTS = 2 * ROWS_PER_W
SLOT_PAD = 32
LANES = 16


def _route_body(x_ref, mb_ref, mfull_ref, wr_ref, g_ref, w_ref, acc_ref):
    s = pl.program_id(0)
    xb = x_ref[...]
    mb = mb_ref[...].astype(jnp.float32)
    part = jnp.sum(xb * mb[:, :, None], axis=1)

    @pl.when(s == 0)
    def _init():
        acc_ref[...] = part

    @pl.when(s > 0)
    def _accum():
        acc_ref[...] = acc_ref[...] + part

    @pl.when(s == NS_BLK - 1)
    def _finalize():
        mfull = mfull_ref[...].astype(jnp.float32)
        cnt = jnp.clip(jnp.sum(mfull, axis=1, keepdims=True), 1.0, None)
        sent = acc_ref[...] / cnt
        logits = lax.dot_general(
            sent, wr_ref[...], (((1,), (1,)), ((), ())),
            precision=lax.Precision.HIGHEST,
            preferred_element_type=jnp.float32)
        z = logits - jnp.max(logits, axis=1, keepdims=True)
        ez = jnp.exp(z)
        p = ez / jnp.sum(ez, axis=1, keepdims=True)

        iota = lax.broadcasted_iota(jnp.int32, (B, N_ROUTES), 1)
        m1 = jnp.max(p, axis=1, keepdims=True)
        i1 = jnp.min(jnp.where(p == m1, iota, N_ROUTES), axis=1, keepdims=True)
        p2 = jnp.where(iota == i1, -1.0, p)
        m2 = jnp.max(p2, axis=1, keepdims=True)
        i2 = jnp.min(jnp.where(p2 == m2, iota, N_ROUTES), axis=1, keepdims=True)

        wq = lax.broadcasted_iota(jnp.int32, (NW, SLOT_PAD), 0)
        sq = lax.broadcasted_iota(jnp.int32, (NW, SLOT_PAD), 1)
        lq = sq % LANES
        route0 = sq < LANES
        r = wq * ROWS_PER_W + lq
        bq = jnp.minimum(r // NVT, B - 1)
        jq = r % NVT
        valid = (lq < ROWS_PER_W) & (r < B * NVT)
        gsel = jnp.zeros((NW, SLOT_PAD), jnp.int32)
        wsel = jnp.zeros((NW, SLOT_PAD), jnp.float32)
        for bb in range(B):
            onb = bq == bb
            t1 = lax.slice(i1, (bb, 0), (bb + 1, 1))
            t2 = lax.slice(i2, (bb, 0), (bb + 1, 1))
            v1 = lax.slice(m1, (bb, 0), (bb + 1, 1))
            v2 = lax.slice(m2, (bb, 0), (bb + 1, 1))
            gsel = gsel + jnp.where(onb, jnp.where(route0, t1, t2), 0)
            wsel = wsel + jnp.where(onb, jnp.where(route0, v1, v2), 0.0)
        g_ref[...] = gsel * NVT + jq
        w_ref[...] = jnp.where(valid, wsel, 0.0)


_route = pl.pallas_call(
    _route_body,
    grid=(NS_BLK,),
    in_specs=[
        pl.BlockSpec((B, S_BLK, D), lambda s: (0, s, 0)),
        pl.BlockSpec((B, S_BLK), lambda s: (0, s)),
        pl.BlockSpec((B, S), lambda s: (0, 0)),
        pl.BlockSpec((N_ROUTES, D), lambda s: (0, 0)),
    ],
    out_specs=[
        pl.BlockSpec((NW, SLOT_PAD), lambda s: (0, 0)),
        pl.BlockSpec((NW, SLOT_PAD), lambda s: (0, 0)),
    ],
    out_shape=[
        jax.ShapeDtypeStruct((NW, SLOT_PAD), jnp.int32),
        jax.ShapeDtypeStruct((NW, SLOT_PAD), jnp.float32),
    ],
    scratch_shapes=[pltpu.VMEM((B, D), jnp.float32)],
)


def _combine_body(emb_ref, g_ref, w_ref, out_ref, g_v, w_v, rows_v, out_v, sem):
    wid = lax.axis_index("s") * NC + lax.axis_index("c")
    pltpu.sync_copy(g_ref, g_v)
    pltpu.sync_copy(w_ref, w_v)
    pltpu.async_copy(emb_ref.at[g_v.at[wid]], rows_v, sem).wait()
    wa = w_v[wid, pl.ds(0, LANES)]
    wb = w_v[wid, pl.ds(LANES, LANES)]

    dnums = lax.GatherDimensionNumbers(
        offset_dims=(), collapsed_slice_dims=(0,), start_index_map=(0,))

    def row_body(i, carry):
        iv = jnp.full((LANES, 1), i, jnp.int32)
        w0 = lax.gather(wa, iv, dnums, (1,),
                        mode=lax.GatherScatterMode.PROMISE_IN_BOUNDS)
        w1 = lax.gather(wb, iv, dnums, (1,),
                        mode=lax.GatherScatterMode.PROMISE_IN_BOUNDS)

        @plsc.parallel_loop(0, D, step=LANES, unroll=8)
        def _col_loop(c):
            a = rows_v[i, pl.ds(c, LANES)]
            b2 = rows_v[i + LANES, pl.ds(c, LANES)]
            out_v[i, pl.ds(c, LANES)] = a * w0 + b2 * w1

        return carry

    lax.fori_loop(0, ROWS_PER_W, row_body, 0)
    pltpu.sync_copy(out_v, out_ref.at[wid])


@functools.cache
def _get_combine():
    return pl.kernel(
        _combine_body,
        out_type=jax.ShapeDtypeStruct((NW, ROWS_PER_W, D), jnp.float32),
        mesh=plsc.VectorSubcoreMesh(core_axis_name="c", subcore_axis_name="s",
                                    num_cores=NC, num_subcores=NSUB),
        scratch_types=[
            pltpu.VMEM((NW, SLOT_PAD), jnp.int32),
            pltpu.VMEM((NW, SLOT_PAD), jnp.float32),
            pltpu.VMEM((SLOT_PAD, D), jnp.float32),
            pltpu.VMEM((ROWS_PER_W, D), jnp.float32),
            pltpu.SemaphoreType.DMA,
        ],
    )


def kernel(indices, input_ids, inputs_embeds, attention_mask, embedding, W_router):
    g_tab, w_tab = _route(inputs_embeds, attention_mask, attention_mask, W_router)
    out = _get_combine()(embedding, g_tab, w_tab)
    return out.reshape(ROWS_PAD, D)[: B * NVT].reshape(B, NVT, D)
